# baseline (speedup 1.0000x reference)
import functools

import jax
import jax.numpy as jnp
from jax import lax
from jax.experimental import pallas as pl
from jax.experimental.pallas import tpu as pltpu
from jax.experimental.pallas import tpu_sc as plsc

NC = 2
NS = 16
LANES = 16
FSPLIT = 2
K = 16
R = 5
ALIGN = NS * R * K


@functools.lru_cache(maxsize=None)
def _make_msg_agg(N, E, D, EP):
    DH = D // FSPLIT
    assert DH % LANES == 0
    HALF = N // NC
    assert HALF * NC == N
    IDXL = 3
    LOOK = 2
    DRAIN = 2
    ZG = 16 * NS
    ACC_ROWS = ((HALF + 1 + ZG - 1) // ZG) * ZG
    ZCH = ACC_ROWS // ZG
    WB = (HALF // NS) & ~7
    WREM = HALF - WB * NS

    mesh = plsc.VectorSubcoreMesh(core_axis_name="c", subcore_axis_name="s",
                                  num_cores=NC, num_subcores=NS)

    @functools.partial(
        pl.kernel,
        out_type=jax.ShapeDtypeStruct((N, D), jnp.float32),
        mesh=mesh,
        scratch_types=[
            [pltpu.VMEM((K, DH), jnp.float32) for _ in range(R)],
            [pltpu.VMEM((K, DH), jnp.float32) for _ in range(R)],
            [pltpu.VMEM((K,), jnp.int32) for _ in range(R)],
            [pltpu.VMEM((K,), jnp.int32) for _ in range(R)],
            [pltpu.VMEM((K,), jnp.int32) for _ in range(R)],
            pltpu.VMEM((16, DH), jnp.float32),
            pltpu.VMEM((6, LANES), jnp.float32),
            pltpu.VMEM_SHARED((ACC_ROWS, DH), jnp.float32),
            [pltpu.SemaphoreType.DMA for _ in range(R)],
            [pltpu.SemaphoreType.DMA for _ in range(R)],
            [pltpu.SemaphoreType.DMA for _ in range(R)],
        ],
    )
    def msg_agg(xlo_hbm, xhi_hbm, srcp_hbm, dstl_hbm, perm_hbm, ea_hbm,
                meta_hbm, out_hbm,
                xbufs, ebufs, isrc, idst, iper, zrow, metav, acc,
                isems, lsems, ssems):
        c = lax.axis_index("c")
        s = lax.axis_index("s")

        pltpu.sync_copy(meta_hbm, metav)

        def _scalar(row):
            return metav[row, :][0].astype(jnp.int32)

        a0 = _scalar(0)
        a1 = _scalar(1)
        b0 = _scalar(2)
        b1 = _scalar(3)
        mybase = a0 * c
        l0 = _scalar(4)
        l1 = _scalar(5)
        blocks = b0 + (b1 - b0) * c
        chunks = blocks * R
        len16 = l0 + (l1 - l0) * c
        estart = pl.multiple_of(mybase + s * len16, 16)

        def zero_row(j, _):
            for t in range(DH // LANES):
                zrow[j, pl.ds(t * LANES, LANES)] = jnp.zeros((LANES,), jnp.float32)
            return 0
        lax.fori_loop(0, 16, zero_row, 0)

        for f, xh_hbm in ((0, xlo_hbm), (1, xhi_hbm)):
            fbase = f * DH

            def issue_idx(ci, slot):
                base = pl.multiple_of(estart + ci * K, 16)
                pltpu.async_copy(srcp_hbm.at[pl.ds(base, K)], isrc[slot],
                                 isems[slot])
                pltpu.async_copy(dstl_hbm.at[pl.ds(base, K)], idst[slot],
                                 isems[slot])
                pltpu.async_copy(perm_hbm.at[pl.ds(base, K)], iper[slot],
                                 isems[slot])

            def wait_idx(slot):
                for buf in (isrc, idst, iper):
                    pltpu.make_async_copy(srcp_hbm.at[pl.ds(0, K)], buf[slot],
                                          isems[slot]).wait()

            def issue_loads(slot, xh_hbm=xh_hbm, fbase=fbase):
                pltpu.async_copy(xh_hbm.at[isrc[slot]], xbufs[slot],
                                 lsems[slot])
                pltpu.async_copy(ea_hbm.at[iper[slot], pl.ds(fbase, DH)],
                                 ebufs[slot], lsems[slot])

            def wait_loads(slot, xh_hbm=xh_hbm, fbase=fbase):
                pltpu.make_async_copy(xh_hbm.at[pl.ds(0, K)], xbufs[slot],
                                      lsems[slot]).wait()
                pltpu.make_async_copy(ea_hbm.at[pl.ds(0, K), pl.ds(fbase, DH)],
                                      ebufs[slot], lsems[slot]).wait()

            def issue_scatter(slot):
                pltpu.async_copy(xbufs[slot], acc.at[idst[slot]],
                                 ssems[slot], add=True)

            def wait_scatter(slot):
                pltpu.make_async_copy(xbufs[slot], acc.at[pl.ds(0, K)],
                                      ssems[slot]).wait()

            def zero_acc(i, _):
                pltpu.sync_copy(zrow, acc.at[pl.ds(s * (ZCH * 16) + i * 16, 16)])
                return 0
            lax.fori_loop(0, ZCH, zero_acc, 0)
            plsc.subcore_barrier()

            @pl.when(blocks > 0)
            def _():
                for j in range(IDXL):
                    issue_idx(j, j)
                for j in range(LOOK):
                    wait_idx(j)
                    issue_loads(j)

            def block(i, _):
                for b in range(R):
                    ci = i * R + b
                    wait_loads(b)

                    def row(j, _, b=b):
                        for t in range(DH // LANES):
                            sl = pl.ds(t * LANES, LANES)
                            xbufs[b][j, sl] = jnp.maximum(
                                xbufs[b][j, sl] + ebufs[b][j, sl], 0.0)
                        return 0
                    lax.fori_loop(0, K, row, 0)
                    issue_scatter(b)

                    @pl.when(ci >= DRAIN)
                    def _(b=b):
                        wait_scatter((b - DRAIN) % R)

                    @pl.when(ci + IDXL < chunks)
                    def _(ci=ci, b=b):
                        issue_idx(ci + IDXL, (b + IDXL) % R)

                    @pl.when(ci + LOOK < chunks)
                    def _(b=b):
                        wait_idx((b + LOOK) % R)
                        issue_loads((b + LOOK) % R)
                return 0
            lax.fori_loop(0, blocks, block, 0)

            @pl.when(blocks > 0)
            def _():
                for tail in range(DRAIN):
                    wait_scatter((R - DRAIN + tail) % R)

            plsc.subcore_barrier()

            nbase = c * HALF
            pltpu.sync_copy(acc.at[pl.ds(s * WB, WB)],
                            out_hbm.at[pl.ds(nbase + s * WB, WB),
                                       pl.ds(fbase, DH)])
            if WREM > 0:
                @pl.when(s == 0)
                def _(fbase=fbase, nbase=nbase):
                    pltpu.sync_copy(
                        acc.at[pl.ds(NS * WB, WREM)],
                        out_hbm.at[pl.ds(nbase + NS * WB, WREM),
                                   pl.ds(fbase, DH)])
            if f + 1 < FSPLIT:
                plsc.subcore_barrier()

    return msg_agg


@functools.lru_cache(maxsize=None)
def _make_mlp(N, D, last):
    BN = 400
    assert N % BN == 0

    def body(x_ref, agg_ref, w1_ref, b1_ref, w2_ref, b2_ref, o_ref):
        a = agg_ref[...] + x_ref[...]
        h = jnp.dot(a, w1_ref[...], preferred_element_type=jnp.float32,
                    precision=lax.Precision.HIGHEST)
        h = jnp.maximum(h + b1_ref[...], 0.0)
        h = jnp.dot(h, w2_ref[...], preferred_element_type=jnp.float32,
                    precision=lax.Precision.HIGHEST)
        h = h + b2_ref[...]
        if not last:
            h = jnp.maximum(h, 0.0)
        o_ref[...] = h + x_ref[...]

    return pl.pallas_call(
        body,
        out_shape=jax.ShapeDtypeStruct((N, D), jnp.float32),
        grid=(N // BN,),
        in_specs=[
            pl.BlockSpec((BN, D), lambda i: (i, 0)),
            pl.BlockSpec((BN, D), lambda i: (i, 0)),
            pl.BlockSpec((D, D), lambda i: (0, 0)),
            pl.BlockSpec((1, D), lambda i: (0, 0)),
            pl.BlockSpec((D, D), lambda i: (0, 0)),
            pl.BlockSpec((1, D), lambda i: (0, 0)),
        ],
        out_specs=pl.BlockSpec((BN, D), lambda i: (i, 0)),
    )


def kernel(node_attr, edge_index, edge_attr, W1, b1, W2, b2):
    N, D = node_attr.shape
    E = edge_attr.shape[0]
    L = W1.shape[0]
    HALF = N // NC
    EP = E + NC * ALIGN
    src = edge_index[0]
    dst = edge_index[1]

    key = (dst >= HALF).astype(jnp.int32)
    r1 = jnp.cumsum(key)
    r0 = jnp.arange(1, E + 1, dtype=jnp.int32) - r1
    e0 = r0[-1]
    a0 = ((e0 + ALIGN - 1) // ALIGN) * ALIGN
    a1 = (((E - e0) + ALIGN - 1) // ALIGN) * ALIGN
    pos = jnp.where(key == 0, r0 - 1, a0 + r1 - 1)
    srcp = jnp.zeros((EP,), jnp.int32).at[pos].set(src)
    dstl = jnp.full((EP,), HALF, jnp.int32).at[pos].set(dst - key * HALF)
    perm = jnp.zeros((EP,), jnp.int32).at[pos].set(
        jnp.arange(E, dtype=jnp.int32))
    meta = jnp.broadcast_to(
        jnp.stack([a0, a1, a0 // ALIGN, a1 // ALIGN,
                   a0 // NS, a1 // NS]).astype(jnp.float32)[:, None],
        (6, LANES))

    msg_agg = _make_msg_agg(N, E, D, EP)
    x = node_attr
    for l in range(L):
        xlo = lax.slice(x, (0, 0), (N, D // 2))
        xhi = lax.slice(x, (0, D // 2), (N, D))
        agg = msg_agg(xlo, xhi, srcp, dstl, perm, edge_attr, meta)
        mlp = _make_mlp(N, D, l == L - 1)
        x = mlp(x, agg, W1[l], b1[l].reshape(1, D), W2[l], b2[l].reshape(1, D))
    return x


# speedup vs baseline: 2.1151x; 2.1151x over previous
import functools

import jax
import jax.numpy as jnp
from jax import lax
from jax.experimental import pallas as pl
from jax.experimental.pallas import tpu as pltpu
from jax.experimental.pallas import tpu_sc as plsc

NS = 16
LANES = 16
FSPLIT = 2


@functools.lru_cache(maxsize=None)
def _make_msg_agg(N, E, D):
    DH = D // FSPLIT
    assert DH % LANES == 0
    PER_TILE = E // NS
    assert PER_TILE * NS == E
    K = 16
    CHUNKS = PER_TILE // K
    assert CHUNKS * K == PER_TILE
    R = 5
    LOOK = 3
    DRAIN = 2
    assert CHUNKS % R == 0 and LOOK + DRAIN <= R
    ZG = 16 * NS
    ACC_ROWS = ((N + ZG - 1) // ZG) * ZG
    ZCH = ACC_ROWS // ZG
    WB = (N // NS) & ~7
    WREM = N - WB * NS

    mesh = plsc.VectorSubcoreMesh(core_axis_name="c", subcore_axis_name="s",
                                  num_cores=1, num_subcores=NS)

    @functools.partial(
        pl.kernel,
        out_type=jax.ShapeDtypeStruct((N, D), jnp.float32),
        mesh=mesh,
        scratch_types=[
            pltpu.VMEM((PER_TILE,), jnp.int32),
            pltpu.VMEM((PER_TILE,), jnp.int32),
            [pltpu.VMEM((K, DH), jnp.float32) for _ in range(R)],
            [pltpu.VMEM((K, DH), jnp.float32) for _ in range(R)],
            pltpu.VMEM((16, DH), jnp.float32),
            pltpu.VMEM_SHARED((ACC_ROWS, DH), jnp.float32),
            [pltpu.SemaphoreType.DMA for _ in range(R)],
            [pltpu.SemaphoreType.DMA for _ in range(R)],
        ],
    )
    def msg_agg(xlo_hbm, xhi_hbm, src_hbm, dst_hbm, ea_hbm, out_hbm,
                src_v, dst_v, xbufs, ebufs, zrow, acc, lsems, ssems):
        s = lax.axis_index("s")
        ebase = s * PER_TILE

        pltpu.sync_copy(src_hbm.at[pl.ds(ebase, PER_TILE)], src_v)
        pltpu.sync_copy(dst_hbm.at[pl.ds(ebase, PER_TILE)], dst_v)

        def zero_row(j, _):
            for t in range(DH // LANES):
                zrow[j, pl.ds(t * LANES, LANES)] = jnp.zeros((LANES,), jnp.float32)
            return 0
        lax.fori_loop(0, 16, zero_row, 0)

        for f, xh_hbm in ((0, xlo_hbm), (1, xhi_hbm)):
            fbase = f * DH

            def issue_loads(ci, slot, xh_hbm=xh_hbm, fbase=fbase):
                pltpu.async_copy(
                    xh_hbm.at[src_v.at[pl.ds(ci * K, K)]], xbufs[slot],
                    lsems[slot])
                pltpu.async_copy(
                    ea_hbm.at[pl.ds(ebase + ci * K, K), pl.ds(fbase, DH)],
                    ebufs[slot], lsems[slot])

            def wait_loads(slot, xh_hbm=xh_hbm, fbase=fbase):
                pltpu.make_async_copy(
                    xh_hbm.at[pl.ds(0, K)], xbufs[slot], lsems[slot]).wait()
                pltpu.make_async_copy(
                    ea_hbm.at[pl.ds(0, K), pl.ds(fbase, DH)], ebufs[slot],
                    lsems[slot]).wait()

            def issue_scatter(ci, slot):
                pltpu.async_copy(
                    xbufs[slot], acc.at[dst_v.at[pl.ds(ci * K, K)]],
                    ssems[slot], add=True)

            def wait_scatter(slot):
                pltpu.make_async_copy(
                    xbufs[slot], acc.at[pl.ds(0, K)], ssems[slot]).wait()

            def zero_acc(i, _):
                pltpu.sync_copy(zrow, acc.at[pl.ds(s * (ZCH * 16) + i * 16, 16)])
                return 0
            lax.fori_loop(0, ZCH, zero_acc, 0)
            plsc.subcore_barrier()

            for slot in range(LOOK):
                issue_loads(slot, slot)

            def block(i, _):
                for b in range(R):
                    ci = i * R + b
                    wait_loads(b)

                    def row(j, _, b=b):
                        for t in range(DH // LANES):
                            sl = pl.ds(t * LANES, LANES)
                            xbufs[b][j, sl] = jnp.maximum(
                                xbufs[b][j, sl] + ebufs[b][j, sl], 0.0)
                        return 0
                    lax.fori_loop(0, K, row, 0)
                    issue_scatter(ci, b)

                    @pl.when(ci >= DRAIN)
                    def _(b=b):
                        wait_scatter((b - DRAIN) % R)

                    @pl.when(ci + LOOK < CHUNKS)
                    def _(ci=ci, b=b):
                        issue_loads(ci + LOOK, (b + LOOK) % R)
                return 0
            lax.fori_loop(0, CHUNKS // R, block, 0)
            for tail in range(DRAIN):
                wait_scatter((CHUNKS - DRAIN + tail) % R)
            plsc.subcore_barrier()

            pltpu.sync_copy(acc.at[pl.ds(s * WB, WB)],
                            out_hbm.at[pl.ds(s * WB, WB), pl.ds(fbase, DH)])
            if WREM > 0:
                @pl.when(s == 0)
                def _():
                    pltpu.sync_copy(
                        acc.at[pl.ds(NS * WB, WREM)],
                        out_hbm.at[pl.ds(NS * WB, WREM), pl.ds(fbase, DH)])
            if f + 1 < FSPLIT:
                plsc.subcore_barrier()

    return msg_agg


@functools.lru_cache(maxsize=None)
def _make_mlp(N, D, last):
    BN = 400
    assert N % BN == 0
    DH = D // 2

    def body(x_ref, agg_ref, w1_ref, b1_ref, w2_ref, b2_ref,
             o_ref, olo_ref, ohi_ref):
        a = agg_ref[...] + x_ref[...]
        h = jnp.dot(a, w1_ref[...], preferred_element_type=jnp.float32,
                    precision=lax.Precision.HIGHEST)
        h = jnp.maximum(h + b1_ref[...], 0.0)
        h = jnp.dot(h, w2_ref[...], preferred_element_type=jnp.float32,
                    precision=lax.Precision.HIGHEST)
        h = h + b2_ref[...]
        if not last:
            h = jnp.maximum(h, 0.0)
        out = h + x_ref[...]
        o_ref[...] = out
        olo_ref[...] = out[:, :DH]
        ohi_ref[...] = out[:, DH:]

    return pl.pallas_call(
        body,
        out_shape=(jax.ShapeDtypeStruct((N, D), jnp.float32),
                   jax.ShapeDtypeStruct((N, DH), jnp.float32),
                   jax.ShapeDtypeStruct((N, DH), jnp.float32)),
        grid=(N // BN,),
        in_specs=[
            pl.BlockSpec((BN, D), lambda i: (i, 0)),
            pl.BlockSpec((BN, D), lambda i: (i, 0)),
            pl.BlockSpec((D, D), lambda i: (0, 0)),
            pl.BlockSpec((1, D), lambda i: (0, 0)),
            pl.BlockSpec((D, D), lambda i: (0, 0)),
            pl.BlockSpec((1, D), lambda i: (0, 0)),
        ],
        out_specs=(pl.BlockSpec((BN, D), lambda i: (i, 0)),
                   pl.BlockSpec((BN, DH), lambda i: (i, 0)),
                   pl.BlockSpec((BN, DH), lambda i: (i, 0))),
    )


def kernel(node_attr, edge_index, edge_attr, W1, b1, W2, b2):
    N, D = node_attr.shape
    E = edge_attr.shape[0]
    L = W1.shape[0]
    src = edge_index[0]
    dst = edge_index[1]
    msg_agg = _make_msg_agg(N, E, D)
    x = node_attr
    xlo = lax.slice(x, (0, 0), (N, D // 2))
    xhi = lax.slice(x, (0, D // 2), (N, D))
    for l in range(L):
        agg = msg_agg(xlo, xhi, src, dst, edge_attr)
        mlp = _make_mlp(N, D, l == L - 1)
        x, xlo, xhi = mlp(x, agg, W1[l], b1[l].reshape(1, D),
                          W2[l], b2[l].reshape(1, D))
    return x


# speedup vs baseline: 2.1311x; 1.0076x over previous
import functools

import jax
import jax.numpy as jnp
from jax import lax
from jax.experimental import pallas as pl
from jax.experimental.pallas import tpu as pltpu
from jax.experimental.pallas import tpu_sc as plsc

NS = 16
LANES = 16
FSPLIT = 2


@functools.lru_cache(maxsize=None)
def _make_msg_agg(N, E, D):
    DH = D // FSPLIT
    assert DH % LANES == 0
    PER_TILE = E // NS
    assert PER_TILE * NS == E
    K = 16
    CHUNKS = PER_TILE // K
    assert CHUNKS * K == PER_TILE
    R = 5
    LOOK = 3
    DRAIN = 2
    assert CHUNKS % R == 0 and LOOK + DRAIN <= R
    ZG = 16 * NS
    ACC_ROWS = ((N + ZG - 1) // ZG) * ZG
    ZCH = ACC_ROWS // ZG
    WB = (N // NS) & ~7
    WREM = N - WB * NS

    mesh = plsc.VectorSubcoreMesh(core_axis_name="c", subcore_axis_name="s",
                                  num_cores=1, num_subcores=NS)

    @functools.partial(
        pl.kernel,
        out_type=jax.ShapeDtypeStruct((N, D), jnp.float32),
        mesh=mesh,
        scratch_types=[
            pltpu.VMEM((PER_TILE,), jnp.int32),
            pltpu.VMEM((PER_TILE,), jnp.int32),
            [pltpu.VMEM((K, DH), jnp.float32) for _ in range(R)],
            [pltpu.VMEM((K, DH), jnp.float32) for _ in range(R)],
            pltpu.VMEM((16, DH), jnp.float32),
            pltpu.VMEM_SHARED((ACC_ROWS, DH), jnp.float32),
            [pltpu.SemaphoreType.DMA for _ in range(R)],
            [pltpu.SemaphoreType.DMA for _ in range(R)],
            pltpu.SemaphoreType.DMA,
        ],
    )
    def msg_agg(xlo_hbm, xhi_hbm, src_hbm, dst_hbm, ea_hbm, out_hbm,
                src_v, dst_v, xbufs, ebufs, zrow, acc, lsems, ssems, zsem):
        s = lax.axis_index("s")
        ebase = s * PER_TILE

        pltpu.sync_copy(src_hbm.at[pl.ds(ebase, PER_TILE)], src_v)
        pltpu.sync_copy(dst_hbm.at[pl.ds(ebase, PER_TILE)], dst_v)

        def zero_row(j, _):
            for t in range(DH // LANES):
                zrow[j, pl.ds(t * LANES, LANES)] = jnp.zeros((LANES,), jnp.float32)
            return 0
        lax.fori_loop(0, 16, zero_row, 0)

        for f, xh_hbm in ((0, xlo_hbm), (1, xhi_hbm)):
            fbase = f * DH

            def issue_loads(ci, slot, xh_hbm=xh_hbm, fbase=fbase):
                pltpu.async_copy(
                    xh_hbm.at[src_v.at[pl.ds(ci * K, K)]], xbufs[slot],
                    lsems[slot])
                pltpu.async_copy(
                    ea_hbm.at[pl.ds(ebase + ci * K, K), pl.ds(fbase, DH)],
                    ebufs[slot], lsems[slot])

            def wait_loads(slot, xh_hbm=xh_hbm, fbase=fbase):
                pltpu.make_async_copy(
                    xh_hbm.at[pl.ds(0, K)], xbufs[slot], lsems[slot]).wait()
                pltpu.make_async_copy(
                    ea_hbm.at[pl.ds(0, K), pl.ds(fbase, DH)], ebufs[slot],
                    lsems[slot]).wait()

            def issue_scatter(ci, slot):
                pltpu.async_copy(
                    xbufs[slot], acc.at[dst_v.at[pl.ds(ci * K, K)]],
                    ssems[slot], add=True)

            def wait_scatter(slot):
                pltpu.make_async_copy(
                    xbufs[slot], acc.at[pl.ds(0, K)], ssems[slot]).wait()

            def zero_acc(i, _):
                pltpu.async_copy(
                    zrow, acc.at[pl.ds(s * (ZCH * 16) + i * 16, 16)], zsem)
                return 0
            lax.fori_loop(0, ZCH, zero_acc, 0)

            def zero_drain(i, _):
                pltpu.make_async_copy(
                    zrow, acc.at[pl.ds(0, 16)], zsem).wait()
                return 0
            lax.fori_loop(0, ZCH, zero_drain, 0)
            plsc.subcore_barrier()

            for slot in range(LOOK):
                issue_loads(slot, slot)

            def block(i, _):
                for b in range(R):
                    ci = i * R + b
                    wait_loads(b)

                    def row(j, _, b=b):
                        for t in range(DH // LANES):
                            sl = pl.ds(t * LANES, LANES)
                            xbufs[b][j, sl] = jnp.maximum(
                                xbufs[b][j, sl] + ebufs[b][j, sl], 0.0)
                        return 0
                    lax.fori_loop(0, K, row, 0)
                    issue_scatter(ci, b)

                    @pl.when(ci >= DRAIN)
                    def _(b=b):
                        wait_scatter((b - DRAIN) % R)

                    @pl.when(ci + LOOK < CHUNKS)
                    def _(ci=ci, b=b):
                        issue_loads(ci + LOOK, (b + LOOK) % R)
                return 0
            lax.fori_loop(0, CHUNKS // R, block, 0)
            for tail in range(DRAIN):
                wait_scatter((CHUNKS - DRAIN + tail) % R)
            plsc.subcore_barrier()

            pltpu.sync_copy(acc.at[pl.ds(s * WB, WB)],
                            out_hbm.at[pl.ds(s * WB, WB), pl.ds(fbase, DH)])
            if WREM > 0:
                @pl.when(s == 0)
                def _():
                    pltpu.sync_copy(
                        acc.at[pl.ds(NS * WB, WREM)],
                        out_hbm.at[pl.ds(NS * WB, WREM), pl.ds(fbase, DH)])
            if f + 1 < FSPLIT:
                plsc.subcore_barrier()

    return msg_agg


@functools.lru_cache(maxsize=None)
def _make_mlp(N, D, last):
    BN = 400
    assert N % BN == 0
    DH = D // 2

    def body(x_ref, agg_ref, w1_ref, b1_ref, w2_ref, b2_ref,
             o_ref, olo_ref, ohi_ref):
        a = agg_ref[...] + x_ref[...]
        h = jnp.dot(a, w1_ref[...], preferred_element_type=jnp.float32,
                    precision=lax.Precision.HIGHEST)
        h = jnp.maximum(h + b1_ref[...], 0.0)
        h = jnp.dot(h, w2_ref[...], preferred_element_type=jnp.float32,
                    precision=lax.Precision.HIGHEST)
        h = h + b2_ref[...]
        if not last:
            h = jnp.maximum(h, 0.0)
        out = h + x_ref[...]
        o_ref[...] = out
        olo_ref[...] = out[:, :DH]
        ohi_ref[...] = out[:, DH:]

    return pl.pallas_call(
        body,
        out_shape=(jax.ShapeDtypeStruct((N, D), jnp.float32),
                   jax.ShapeDtypeStruct((N, DH), jnp.float32),
                   jax.ShapeDtypeStruct((N, DH), jnp.float32)),
        grid=(N // BN,),
        in_specs=[
            pl.BlockSpec((BN, D), lambda i: (i, 0)),
            pl.BlockSpec((BN, D), lambda i: (i, 0)),
            pl.BlockSpec((D, D), lambda i: (0, 0)),
            pl.BlockSpec((1, D), lambda i: (0, 0)),
            pl.BlockSpec((D, D), lambda i: (0, 0)),
            pl.BlockSpec((1, D), lambda i: (0, 0)),
        ],
        out_specs=(pl.BlockSpec((BN, D), lambda i: (i, 0)),
                   pl.BlockSpec((BN, DH), lambda i: (i, 0)),
                   pl.BlockSpec((BN, DH), lambda i: (i, 0))),
    )


def kernel(node_attr, edge_index, edge_attr, W1, b1, W2, b2):
    N, D = node_attr.shape
    E = edge_attr.shape[0]
    L = W1.shape[0]
    src = edge_index[0]
    dst = edge_index[1]
    msg_agg = _make_msg_agg(N, E, D)
    x = node_attr
    xlo = lax.slice(x, (0, 0), (N, D // 2))
    xhi = lax.slice(x, (0, D // 2), (N, D))
    for l in range(L):
        agg = msg_agg(xlo, xhi, src, dst, edge_attr)
        mlp = _make_mlp(N, D, l == L - 1)
        x, xlo, xhi = mlp(x, agg, W1[l], b1[l].reshape(1, D),
                          W2[l], b2[l].reshape(1, D))
    return x


# speedup vs baseline: 2.6064x; 1.2230x over previous
import functools

import jax
import jax.numpy as jnp
from jax import lax
from jax.experimental import pallas as pl
from jax.experimental.pallas import tpu as pltpu
from jax.experimental.pallas import tpu_sc as plsc

NS = 16
LANES = 16
FSPLIT = 2


@functools.lru_cache(maxsize=None)
def _make_msg_agg(N, E, D):
    DH = D // FSPLIT
    assert DH % LANES == 0
    PER_TILE = E // NS
    assert PER_TILE * NS == E
    K = 16
    CHUNKS = PER_TILE // K
    assert CHUNKS * K == PER_TILE
    R = 5
    LOOK = 4
    DRAIN = 1
    assert CHUNKS % R == 0 and LOOK + DRAIN <= R
    ZG = 16 * NS
    ACC_ROWS = ((N + ZG - 1) // ZG) * ZG
    ZCH = ACC_ROWS // ZG
    WB = (N // NS) & ~7
    WREM = N - WB * NS

    mesh = plsc.VectorSubcoreMesh(core_axis_name="c", subcore_axis_name="s",
                                  num_cores=1, num_subcores=NS)

    @functools.partial(
        pl.kernel,
        out_type=jax.ShapeDtypeStruct((N, D), jnp.float32),
        mesh=mesh,
        scratch_types=[
            pltpu.VMEM((PER_TILE,), jnp.int32),
            pltpu.VMEM((PER_TILE,), jnp.int32),
            [pltpu.VMEM((K, DH), jnp.float32) for _ in range(R)],
            [pltpu.VMEM((K, DH), jnp.float32) for _ in range(R)],
            pltpu.VMEM((16, DH), jnp.float32),
            pltpu.VMEM_SHARED((ACC_ROWS, DH), jnp.float32),
            [pltpu.SemaphoreType.DMA for _ in range(R)],
            [pltpu.SemaphoreType.DMA for _ in range(R)],
            pltpu.SemaphoreType.DMA,
        ],
    )
    def msg_agg(xlo_hbm, xhi_hbm, src_hbm, dst_hbm, ea_hbm, out_hbm,
                src_v, dst_v, xbufs, ebufs, zrow, acc, lsems, ssems, zsem):
        s = lax.axis_index("s")
        ebase = s * PER_TILE

        pltpu.sync_copy(src_hbm.at[pl.ds(ebase, PER_TILE)], src_v)
        pltpu.sync_copy(dst_hbm.at[pl.ds(ebase, PER_TILE)], dst_v)

        def zero_row(j, _):
            for t in range(DH // LANES):
                zrow[j, pl.ds(t * LANES, LANES)] = jnp.zeros((LANES,), jnp.float32)
            return 0
        lax.fori_loop(0, 16, zero_row, 0)

        for f, xh_hbm in ((0, xlo_hbm), (1, xhi_hbm)):
            fbase = f * DH

            def issue_loads(ci, slot, xh_hbm=xh_hbm, fbase=fbase):
                pltpu.async_copy(
                    xh_hbm.at[src_v.at[pl.ds(ci * K, K)]], xbufs[slot],
                    lsems[slot])
                pltpu.async_copy(
                    ea_hbm.at[pl.ds(ebase + ci * K, K), pl.ds(fbase, DH)],
                    ebufs[slot], lsems[slot])

            def wait_loads(slot, xh_hbm=xh_hbm, fbase=fbase):
                pltpu.make_async_copy(
                    xh_hbm.at[pl.ds(0, K)], xbufs[slot], lsems[slot]).wait()
                pltpu.make_async_copy(
                    ea_hbm.at[pl.ds(0, K), pl.ds(fbase, DH)], ebufs[slot],
                    lsems[slot]).wait()

            def issue_scatter(ci, slot):
                pltpu.async_copy(
                    xbufs[slot], acc.at[dst_v.at[pl.ds(ci * K, K)]],
                    ssems[slot], add=True)

            def wait_scatter(slot):
                pltpu.make_async_copy(
                    xbufs[slot], acc.at[pl.ds(0, K)], ssems[slot]).wait()

            def zero_acc(i, _):
                pltpu.async_copy(
                    zrow, acc.at[pl.ds(s * (ZCH * 16) + i * 16, 16)], zsem)
                return 0
            lax.fori_loop(0, ZCH, zero_acc, 0)

            def zero_drain(i, _):
                pltpu.make_async_copy(
                    zrow, acc.at[pl.ds(0, 16)], zsem).wait()
                return 0
            lax.fori_loop(0, ZCH, zero_drain, 0)
            plsc.subcore_barrier()

            for slot in range(LOOK):
                issue_loads(slot, slot)

            def block(i, _):
                for b in range(R):
                    ci = i * R + b
                    wait_loads(b)

                    def row(j, _, b=b):
                        for t in range(DH // LANES):
                            sl = pl.ds(t * LANES, LANES)
                            xbufs[b][j, sl] = jnp.maximum(
                                xbufs[b][j, sl] + ebufs[b][j, sl], 0.0)
                        return 0
                    lax.fori_loop(0, K, row, 0)
                    issue_scatter(ci, b)

                    @pl.when(ci >= DRAIN)
                    def _(b=b):
                        wait_scatter((b - DRAIN) % R)

                    @pl.when(ci + LOOK < CHUNKS)
                    def _(ci=ci, b=b):
                        issue_loads(ci + LOOK, (b + LOOK) % R)
                return 0
            lax.fori_loop(0, CHUNKS // R, block, 0)
            for tail in range(DRAIN):
                wait_scatter((CHUNKS - DRAIN + tail) % R)
            plsc.subcore_barrier()

            pltpu.sync_copy(acc.at[pl.ds(s * WB, WB)],
                            out_hbm.at[pl.ds(s * WB, WB), pl.ds(fbase, DH)])
            if WREM > 0:
                @pl.when(s == 0)
                def _():
                    pltpu.sync_copy(
                        acc.at[pl.ds(NS * WB, WREM)],
                        out_hbm.at[pl.ds(NS * WB, WREM), pl.ds(fbase, DH)])
            if f + 1 < FSPLIT:
                plsc.subcore_barrier()

    return msg_agg


@functools.lru_cache(maxsize=None)
def _make_mlp(N, D, last):
    BN = 400
    assert N % BN == 0
    DH = D // 2

    def body(x_ref, agg_ref, w1_ref, b1_ref, w2_ref, b2_ref,
             o_ref, olo_ref, ohi_ref):
        a = agg_ref[...] + x_ref[...]
        h = jnp.dot(a, w1_ref[...], preferred_element_type=jnp.float32,
                    precision=lax.Precision.HIGHEST)
        h = jnp.maximum(h + b1_ref[...], 0.0)
        h = jnp.dot(h, w2_ref[...], preferred_element_type=jnp.float32,
                    precision=lax.Precision.HIGHEST)
        h = h + b2_ref[...]
        if not last:
            h = jnp.maximum(h, 0.0)
        out = h + x_ref[...]
        o_ref[...] = out
        olo_ref[...] = out[:, :DH]
        ohi_ref[...] = out[:, DH:]

    return pl.pallas_call(
        body,
        out_shape=(jax.ShapeDtypeStruct((N, D), jnp.float32),
                   jax.ShapeDtypeStruct((N, DH), jnp.float32),
                   jax.ShapeDtypeStruct((N, DH), jnp.float32)),
        grid=(N // BN,),
        in_specs=[
            pl.BlockSpec((BN, D), lambda i: (i, 0)),
            pl.BlockSpec((BN, D), lambda i: (i, 0)),
            pl.BlockSpec((D, D), lambda i: (0, 0)),
            pl.BlockSpec((1, D), lambda i: (0, 0)),
            pl.BlockSpec((D, D), lambda i: (0, 0)),
            pl.BlockSpec((1, D), lambda i: (0, 0)),
        ],
        out_specs=(pl.BlockSpec((BN, D), lambda i: (i, 0)),
                   pl.BlockSpec((BN, DH), lambda i: (i, 0)),
                   pl.BlockSpec((BN, DH), lambda i: (i, 0))),
    )


def kernel(node_attr, edge_index, edge_attr, W1, b1, W2, b2):
    N, D = node_attr.shape
    E = edge_attr.shape[0]
    L = W1.shape[0]
    src = edge_index[0]
    dst = edge_index[1]
    msg_agg = _make_msg_agg(N, E, D)
    x = node_attr
    xlo = lax.slice(x, (0, 0), (N, D // 2))
    xhi = lax.slice(x, (0, D // 2), (N, D))
    for l in range(L):
        agg = msg_agg(xlo, xhi, src, dst, edge_attr)
        mlp = _make_mlp(N, D, l == L - 1)
        x, xlo, xhi = mlp(x, agg, W1[l], b1[l].reshape(1, D),
                          W2[l], b2[l].reshape(1, D))
    return x
